# baseline (speedup 1.0000x reference)
import functools

import jax
import jax.numpy as jnp
from jax import lax
from jax.experimental import pallas as pl
from jax.experimental.pallas import tpu as pltpu
from jax.experimental.pallas import tpu_sc as plsc

B0, B1 = 16384, 20
D = 128
BP = 24
NC, NS = 2, 16
NW = NC * NS
GPW = B0 // NW
PPW = GPW * BP
G = 8
CWP = G * BP
NISS = 2
IW = CWP // NISS
NCH = GPW // G
NBUF = 4
NG = NCH // NBUF

_mesh = plsc.VectorSubcoreMesh(core_axis_name="c", subcore_axis_name="s")


@functools.partial(
    pl.kernel,
    mesh=_mesh,
    out_type=jax.ShapeDtypeStruct((B0, B1, D), jnp.float32),
    scratch_types=[pltpu.VMEM((PPW,), jnp.int32)]
    + [pltpu.VMEM((CWP, D), jnp.float32) for _ in range(NBUF)]
    + [pltpu.SemaphoreType.DMA for _ in range(2 * NBUF)],
)
def _gather(table_hbm, idx_hbm, out_hbm, idx_v, *bufs_sems):
    bufs = bufs_sems[:NBUF]
    gsem = bufs_sems[NBUF:2 * NBUF]
    osem = bufs_sems[2 * NBUF:]

    wid = lax.axis_index("s") * NC + lax.axis_index("c")
    row_base = pl.multiple_of(wid * PPW, PPW)
    grp_base = pl.multiple_of(wid * GPW, GPW)

    pltpu.sync_copy(idx_hbm.at[pl.ds(row_base, PPW)], idx_v)

    def fire_gather(k, b):
        for j in range(NISS):
            pltpu.async_copy(
                table_hbm.at[idx_v.at[pl.ds(k * CWP + j * IW, IW)]],
                bufs[b].at[pl.ds(j * IW, IW)],
                gsem[b],
            )

    def wait_gather(k, b):
        for j in range(NISS):
            pltpu.make_async_copy(
                table_hbm.at[idx_v.at[pl.ds(k * CWP + j * IW, IW)]],
                bufs[b].at[pl.ds(j * IW, IW)],
                gsem[b],
            ).wait()

    def fire_wb(k, b):
        for g in range(G):
            pltpu.async_copy(
                bufs[b].at[pl.ds(g * BP, B1)],
                out_hbm.at[grp_base + k * G + g],
                osem[b],
            )

    def wait_wb(k, b):
        for g in range(G):
            pltpu.make_async_copy(
                bufs[b].at[pl.ds(g * BP, B1)],
                out_hbm.at[grp_base + k * G + g],
                osem[b],
            ).wait()

    for b in range(NBUF):
        fire_gather(b, b)

    def ring_round(p, carry):
        for b in range(NBUF):
            k = p * NBUF + b
            wait_gather(k, b)
            fire_wb(k, b)
        for b in range(NBUF):
            k = p * NBUF + b
            wait_wb(k, b)
            fire_gather(k + NBUF, b)
        return carry

    lax.fori_loop(0, NG - 1, ring_round, 0)

    for b in range(NBUF):
        k = (NG - 1) * NBUF + b
        wait_gather(k, b)
        fire_wb(k, b)
    for b in range(NBUF):
        k = (NG - 1) * NBUF + b
        wait_wb(k, b)


def kernel(id, ttaEncoding):
    idx_pad = jnp.pad(id.astype(jnp.int32), ((0, 0), (0, BP - B1)), mode="edge")
    return _gather(ttaEncoding, idx_pad.reshape(B0 * BP))


# speedup vs baseline: 1.1244x; 1.1244x over previous
import functools

import jax
import jax.numpy as jnp
from jax import lax
from jax.experimental import pallas as pl
from jax.experimental.pallas import tpu as pltpu
from jax.experimental.pallas import tpu_sc as plsc

B0, B1 = 16384, 20
D = 128
B = B0 * B1
NC, NS = 2, 16
NW = NC * NS
BPW = B // NW
GPW = B0 // NW
G = 4
CW = G * B1
NCH = GPW // G
NBUF = 4
NG = NCH // NBUF

_mesh = plsc.VectorSubcoreMesh(core_axis_name="c", subcore_axis_name="s")


@functools.partial(
    pl.kernel,
    mesh=_mesh,
    out_type=jax.ShapeDtypeStruct((B0, B1, D), jnp.float32),
    scratch_types=[pltpu.VMEM((GPW, B1), jnp.int32)]
    + [pltpu.VMEM((G, B1, D), jnp.float32) for _ in range(NBUF)]
    + [pltpu.SemaphoreType.DMA for _ in range(2 * NBUF)],
)
def _gather(table_hbm, idx_hbm, out_hbm, idx_v, *bufs_sems):
    bufs = bufs_sems[:NBUF]
    gsem = bufs_sems[NBUF:2 * NBUF]
    osem = bufs_sems[2 * NBUF:]

    wid = lax.axis_index("s") * NC + lax.axis_index("c")
    grp_base = pl.multiple_of(wid * GPW, GPW)

    pltpu.sync_copy(idx_hbm.at[pl.ds(grp_base, GPW)], idx_v)

    def fire_gather(k, b):
        for g in range(G):
            pltpu.async_copy(
                table_hbm.at[idx_v.at[k * G + g]],
                bufs[b].at[g],
                gsem[b],
            )

    def wait_gather(k, b):
        for g in range(G):
            pltpu.make_async_copy(
                table_hbm.at[idx_v.at[k * G + g]],
                bufs[b].at[g],
                gsem[b],
            ).wait()

    def fire_wb(k, b):
        pltpu.async_copy(bufs[b], out_hbm.at[pl.ds(grp_base + k * G, G)], osem[b])

    def wait_wb(k, b):
        pltpu.make_async_copy(
            bufs[b], out_hbm.at[pl.ds(grp_base + k * G, G)], osem[b]
        ).wait()

    for b in range(NBUF):
        fire_gather(b, b)

    def ring_round(p, carry):
        for b in range(NBUF):
            k = p * NBUF + b
            wait_gather(k, b)
            fire_wb(k, b)
            wait_wb(k, b)
            fire_gather(k + NBUF, b)
        return carry

    lax.fori_loop(0, NG - 1, ring_round, 0)

    for b in range(NBUF):
        k = (NG - 1) * NBUF + b
        wait_gather(k, b)
        fire_wb(k, b)
    for b in range(NBUF):
        k = (NG - 1) * NBUF + b
        wait_wb(k, b)


def kernel(id, ttaEncoding):
    return _gather(ttaEncoding, id.astype(jnp.int32))


# speedup vs baseline: 2.1262x; 1.8910x over previous
import functools

import jax
import jax.numpy as jnp
from jax import lax
from jax.experimental import pallas as pl
from jax.experimental.pallas import tpu as pltpu
from jax.experimental.pallas import tpu_sc as plsc

B0, B1 = 16384, 20
D = 128
B = B0 * B1
NC, NS = 2, 16
NW = NC * NS
BPW = B // NW
IW = 128
IRPW = BPW // IW
CW = 256
NISS = CW // IW
NCH = BPW // CW
NBUF = 2
NG = NCH // NBUF

_mesh = plsc.VectorSubcoreMesh(core_axis_name="c", subcore_axis_name="s")


@functools.partial(
    pl.kernel,
    mesh=_mesh,
    out_type=jax.ShapeDtypeStruct((B1, B0, D), jnp.float32),
    scratch_types=[pltpu.VMEM((IRPW, IW), jnp.int32)]
    + [pltpu.VMEM((CW, D), jnp.float32) for _ in range(NBUF)]
    + [pltpu.SemaphoreType.DMA for _ in range(2 * NBUF)],
)
def _gather(table_hbm, idx_hbm, out_hbm, idx_v, *bufs_sems):
    bufs = bufs_sems[:NBUF]
    gsem = bufs_sems[NBUF:2 * NBUF]
    osem = bufs_sems[2 * NBUF:]

    wid = lax.axis_index("s") * NC + lax.axis_index("c")
    row_base = pl.multiple_of(wid * BPW, BPW)
    irow_base = pl.multiple_of(wid * IRPW, IRPW)

    pltpu.sync_copy(idx_hbm.at[pl.ds(irow_base, IRPW)], idx_v)

    def fire_gather(k, b):
        for j in range(NISS):
            pltpu.async_copy(
                table_hbm.at[idx_v.at[k * NISS + j]],
                bufs[b].at[pl.ds(j * IW, IW)],
                gsem[b],
            )

    def wait_gather(k, b):
        for j in range(NISS):
            pltpu.make_async_copy(
                table_hbm.at[idx_v.at[k * NISS + j]],
                bufs[b].at[pl.ds(j * IW, IW)],
                gsem[b],
            ).wait()

    def _dst(k):
        off = row_base + k * CW
        p = off // B0
        col = pl.multiple_of(off - p * B0, CW)
        return out_hbm.at[p, pl.ds(col, CW)]

    def fire_wb(k, b):
        pltpu.async_copy(bufs[b], _dst(k), osem[b])

    def wait_wb(k, b):
        pltpu.make_async_copy(bufs[b], _dst(k), osem[b]).wait()

    for b in range(NBUF):
        fire_gather(b, b)

    def ring_round(p, carry):
        for b in range(NBUF):
            k = p * NBUF + b
            wait_gather(k, b)
            fire_wb(k, b)
            wait_wb(k, b)
            fire_gather(k + NBUF, b)
        return carry

    lax.fori_loop(0, NG - 1, ring_round, 0)

    for b in range(NBUF):
        k = (NG - 1) * NBUF + b
        wait_gather(k, b)
        fire_wb(k, b)
    for b in range(NBUF):
        k = (NG - 1) * NBUF + b
        wait_wb(k, b)


def kernel(id, ttaEncoding):
    idx2 = jnp.swapaxes(id.astype(jnp.int32), 0, 1).reshape(B // IW, IW)
    out = _gather(ttaEncoding, idx2)
    return jnp.swapaxes(out, 0, 1)


# speedup vs baseline: 2.1322x; 1.0028x over previous
import functools

import jax
import jax.numpy as jnp
from jax import lax
from jax.experimental import pallas as pl
from jax.experimental.pallas import tpu as pltpu
from jax.experimental.pallas import tpu_sc as plsc

B0, B1 = 16384, 20
D = 128
B = B0 * B1
NC, NS = 2, 16
NW = NC * NS
BPW = B // NW
IW = 128
IRPW = BPW // IW
CW = 128
NISS = CW // IW
NCH = BPW // CW
NBUF = 4
NG = NCH // NBUF

_mesh = plsc.VectorSubcoreMesh(core_axis_name="c", subcore_axis_name="s")


@functools.partial(
    pl.kernel,
    mesh=_mesh,
    out_type=jax.ShapeDtypeStruct((B1, B0, D), jnp.float32),
    scratch_types=[pltpu.VMEM((IRPW, IW), jnp.int32)]
    + [pltpu.VMEM((CW, D), jnp.float32) for _ in range(NBUF)]
    + [pltpu.SemaphoreType.DMA for _ in range(2 * NBUF)],
)
def _gather(table_hbm, idx_hbm, out_hbm, idx_v, *bufs_sems):
    bufs = bufs_sems[:NBUF]
    gsem = bufs_sems[NBUF:2 * NBUF]
    osem = bufs_sems[2 * NBUF:]

    wid = lax.axis_index("s") * NC + lax.axis_index("c")
    row_base = pl.multiple_of(wid * BPW, BPW)
    irow_base = pl.multiple_of(wid * IRPW, IRPW)

    pltpu.sync_copy(idx_hbm.at[pl.ds(irow_base, IRPW)], idx_v)

    def fire_gather(k, b):
        for j in range(NISS):
            pltpu.async_copy(
                table_hbm.at[idx_v.at[k * NISS + j]],
                bufs[b].at[pl.ds(j * IW, IW)],
                gsem[b],
            )

    def wait_gather(k, b):
        for j in range(NISS):
            pltpu.make_async_copy(
                table_hbm.at[idx_v.at[k * NISS + j]],
                bufs[b].at[pl.ds(j * IW, IW)],
                gsem[b],
            ).wait()

    def _dst(k):
        off = row_base + k * CW
        p = off // B0
        col = pl.multiple_of(off - p * B0, CW)
        return out_hbm.at[p, pl.ds(col, CW)]

    def fire_wb(k, b):
        pltpu.async_copy(bufs[b], _dst(k), osem[b])

    def wait_wb(k, b):
        pltpu.make_async_copy(bufs[b], _dst(k), osem[b]).wait()

    for b in range(NBUF):
        fire_gather(b, b)

    def ring_round(p, carry):
        for b in range(NBUF):
            k = p * NBUF + b
            wait_gather(k, b)
            fire_wb(k, b)
            wait_wb(k, b)
            fire_gather(k + NBUF, b)
        return carry

    lax.fori_loop(0, NG - 1, ring_round, 0)

    for b in range(NBUF):
        k = (NG - 1) * NBUF + b
        wait_gather(k, b)
        fire_wb(k, b)
    for b in range(NBUF):
        k = (NG - 1) * NBUF + b
        wait_wb(k, b)


def kernel(id, ttaEncoding):
    idx2 = jnp.swapaxes(id.astype(jnp.int32), 0, 1).reshape(B // IW, IW)
    out = _gather(ttaEncoding, idx2)
    return jnp.swapaxes(out, 0, 1)


# speedup vs baseline: 2.1327x; 1.0002x over previous
import functools

import jax
import jax.numpy as jnp
from jax import lax
from jax.experimental import pallas as pl
from jax.experimental.pallas import tpu as pltpu
from jax.experimental.pallas import tpu_sc as plsc

B0, B1 = 16384, 20
D = 128
B = B0 * B1
NC, NS = 2, 16
NW = NC * NS
BPW = B // NW
IW = 128
IRPW = BPW // IW
CW = 128
NISS = CW // IW
NCH = BPW // CW
NBUF = 5
NG = NCH // NBUF

_mesh = plsc.VectorSubcoreMesh(core_axis_name="c", subcore_axis_name="s")


@functools.partial(
    pl.kernel,
    mesh=_mesh,
    out_type=jax.ShapeDtypeStruct((B1, B0, D), jnp.float32),
    scratch_types=[pltpu.VMEM((IRPW, IW), jnp.int32)]
    + [pltpu.VMEM((CW, D), jnp.float32) for _ in range(NBUF)]
    + [pltpu.SemaphoreType.DMA for _ in range(2 * NBUF)],
)
def _gather(table_hbm, idx_hbm, out_hbm, idx_v, *bufs_sems):
    bufs = bufs_sems[:NBUF]
    gsem = bufs_sems[NBUF:2 * NBUF]
    osem = bufs_sems[2 * NBUF:]

    wid = lax.axis_index("s") * NC + lax.axis_index("c")
    row_base = pl.multiple_of(wid * BPW, BPW)
    irow_base = pl.multiple_of(wid * IRPW, IRPW)

    pltpu.sync_copy(idx_hbm.at[pl.ds(irow_base, IRPW)], idx_v)

    def fire_gather(k, b):
        for j in range(NISS):
            pltpu.async_copy(
                table_hbm.at[idx_v.at[k * NISS + j]],
                bufs[b].at[pl.ds(j * IW, IW)],
                gsem[b],
            )

    def wait_gather(k, b):
        for j in range(NISS):
            pltpu.make_async_copy(
                table_hbm.at[idx_v.at[k * NISS + j]],
                bufs[b].at[pl.ds(j * IW, IW)],
                gsem[b],
            ).wait()

    def _dst(k):
        off = row_base + k * CW
        p = off // B0
        col = pl.multiple_of(off - p * B0, CW)
        return out_hbm.at[p, pl.ds(col, CW)]

    def fire_wb(k, b):
        pltpu.async_copy(bufs[b], _dst(k), osem[b])

    def wait_wb(k, b):
        pltpu.make_async_copy(bufs[b], _dst(k), osem[b]).wait()

    for b in range(NBUF):
        fire_gather(b, b)

    def ring_round(p, carry):
        for b in range(NBUF):
            k = p * NBUF + b
            wait_gather(k, b)
            fire_wb(k, b)
            wait_wb(k, b)
            fire_gather(k + NBUF, b)
        return carry

    lax.fori_loop(0, NG - 1, ring_round, 0)

    for b in range(NBUF):
        k = (NG - 1) * NBUF + b
        wait_gather(k, b)
        fire_wb(k, b)
    for b in range(NBUF):
        k = (NG - 1) * NBUF + b
        wait_wb(k, b)


def kernel(id, ttaEncoding):
    idx2 = jnp.swapaxes(id.astype(jnp.int32), 0, 1).reshape(B // IW, IW)
    out = _gather(ttaEncoding, idx2)
    return jnp.swapaxes(out, 0, 1)
